# baseline (speedup 1.0000x reference)
import functools

import jax
import jax.numpy as jnp
from jax import lax
from jax.experimental import pallas as pl
from jax.experimental.pallas import tpu as pltpu
from jax.experimental.pallas import tpu_sc as plsc

F32 = jnp.float32

NC = 2
NS = 16
NW = NC * NS

N_NODES = 10000
DIM = 128
E = 320000
EPW = E // NW
CH = 80
NCH = EPW // CH
ROWS_PS = 640
BDEC = 8192
DEC_PW = BDEC // NW


def _sds(shape, dtype=F32):
  return jax.ShapeDtypeStruct(shape, dtype)


def _row_start(sid):
  return lax.min(sid * ROWS_PS, N_NODES - ROWS_PS)


def _sc_degrees(src, dst):
  mesh = plsc.VectorSubcoreMesh(core_axis_name="c", subcore_axis_name="s")

  @functools.partial(
      pl.kernel,
      out_type=(_sds((NW, N_NODES)), _sds((NW, N_NODES))),
      mesh=mesh,
      scratch_types=[
          pltpu.VMEM((EPW,), jnp.int32),
          pltpu.VMEM((EPW,), jnp.int32),
          pltpu.VMEM((N_NODES,), F32),
          pltpu.VMEM((N_NODES,), F32),
      ],
      compiler_params=pltpu.CompilerParams(needs_layout_passes=False),
  )
  def deg_kernel(src_hbm, dst_hbm, dout_hbm, din_hbm, sidx, didx, ho, hi):
    cid = lax.axis_index("c")
    sid = lax.axis_index("s")
    wid = sid * NC + cid
    pltpu.sync_copy(src_hbm.at[pl.ds(wid * EPW, EPW)], sidx)
    pltpu.sync_copy(dst_hbm.at[pl.ds(wid * EPW, EPW)], didx)

    zero16 = jnp.zeros((16,), F32)

    def zbody(i, _):
      ho[pl.ds(i * 16, 16)] = zero16
      hi[pl.ds(i * 16, 16)] = zero16
      return ()

    lax.fori_loop(0, N_NODES // 16, zbody, ())

    ones16 = jnp.ones((16,), F32)

    def body(i, _):
      plsc.addupdate_scatter(ho, [sidx[pl.ds(i * 16, 16)]], ones16)
      plsc.addupdate_scatter(hi, [didx[pl.ds(i * 16, 16)]], ones16)
      return ()

    lax.fori_loop(0, EPW // 16, body, ())
    pltpu.sync_copy(ho, dout_hbm.at[wid])
    pltpu.sync_copy(hi, din_hbm.at[wid])

  return deg_kernel(src, dst)


CHP = 104
EPP = 10192
NCHP = EPP // CHP
N_PAD = N_NODES + 16


def _sc_aggregate(node_pad, srcp, dstp, znode):
  mesh = plsc.VectorSubcoreMesh(core_axis_name="c", subcore_axis_name="s")

  @functools.partial(
      pl.kernel,
      out_type=_sds((NC, N_NODES, DIM)),
      mesh=mesh,
      scratch_types=[
          pltpu.VMEM((CHP,), jnp.int32),
          pltpu.VMEM((CHP,), jnp.int32),
          pltpu.VMEM((CHP,), jnp.int32),
          pltpu.VMEM((CHP,), jnp.int32),
          pltpu.VMEM((CHP, DIM), F32),
          pltpu.VMEM_SHARED((N_NODES, DIM), F32),
          pltpu.SemaphoreType.DMA,
          pltpu.SemaphoreType.DMA,
          pltpu.SemaphoreType.DMA,
      ],
  )
  def agg_kernel(nodef, srcp_h, dstp_h, znode_h, out_hbm,
                 sidx_a, didx_a, sidx_b, didx_b, rows, acc, sem, is_a, is_b):
    cid = lax.axis_index("c")
    sid = lax.axis_index("s")
    wid = sid * NC + cid
    rs = _row_start(sid)
    pltpu.sync_copy(znode_h.at[pl.ds(rs, ROWS_PS)], acc.at[pl.ds(rs, ROWS_PS)])
    plsc.subcore_barrier()

    ebase = wid * EPP
    last_off = ebase + (NCHP - 1) * CHP

    def fire_pair(off, si, di, s):
      pltpu.async_copy(srcp_h.at[pl.ds(off, CHP)], si, s)
      pltpu.async_copy(dstp_h.at[pl.ds(off, CHP)], di, s)

    def wait_pair(off, si, di, s):
      pltpu.make_async_copy(srcp_h.at[pl.ds(off, CHP)], si, s).wait()
      pltpu.make_async_copy(dstp_h.at[pl.ds(off, CHP)], di, s).wait()

    fire_pair(ebase, sidx_a, didx_a, is_a)

    def body(j, _):
      off_e = ebase + (2 * j) * CHP
      off_o = off_e + CHP
      wait_pair(off_e, sidx_a, didx_a, is_a)
      fire_pair(off_o, sidx_b, didx_b, is_b)
      pltpu.async_copy(nodef.at[sidx_a], rows, sem).wait()
      pltpu.sync_copy(rows, acc.at[didx_a], add=True)
      wait_pair(off_o, sidx_b, didx_b, is_b)
      fire_pair(lax.min(off_o + CHP, last_off), sidx_a, didx_a, is_a)
      pltpu.async_copy(nodef.at[sidx_b], rows, sem).wait()
      pltpu.sync_copy(rows, acc.at[didx_b], add=True)
      return ()

    lax.fori_loop(0, NCHP // 2, body, ())
    wait_pair(ebase, sidx_a, didx_a, is_a)
    plsc.subcore_barrier()
    pltpu.sync_copy(acc.at[pl.ds(rs, ROWS_PS)],
                    out_hbm.at[cid, pl.ds(rs, ROWS_PS)])

  return agg_kernel(node_pad, srcp, dstp, znode)


def _sc_decoder_gather(user_embed, item_embed, user_id, pos_id, neg_id):
  mesh = plsc.VectorSubcoreMesh(core_axis_name="c", subcore_axis_name="s")
  CHD = 128
  NCHD = DEC_PW // CHD

  @functools.partial(
      pl.kernel,
      out_type=_sds((3, BDEC, DIM)),
      mesh=mesh,
      scratch_types=[
          pltpu.VMEM((CHD,), jnp.int32),
          pltpu.VMEM((CHD, DIM), F32),
          pltpu.SemaphoreType.DMA,
      ],
  )
  def gather_kernel(ue_hbm, ie_hbm, uid_hbm, pid_hbm, nid_hbm, out_hbm,
                    idx, rows, sem):
    cid = lax.axis_index("c")
    sid = lax.axis_index("s")
    wid = sid * NC + cid
    base = wid * DEC_PW
    for t, (tab, ids) in enumerate(
        ((ue_hbm, uid_hbm), (ie_hbm, pid_hbm), (ie_hbm, nid_hbm))):
      for c in range(NCHD):
        off = base + c * CHD
        pltpu.sync_copy(ids.at[pl.ds(off, CHD)], idx)
        pltpu.async_copy(tab.at[idx], rows, sem).wait()
        pltpu.sync_copy(rows, out_hbm.at[t, pl.ds(off, CHD)])

  return gather_kernel(user_embed, item_embed, user_id, pos_id, neg_id)


def _tc_encoder(uf, itf, Wu, bu, Wi, bi):
  def body(uf_r, it_r, wu_r, bu_r, wi_r, bi_r, ue_o, ie_o):
    ue = jnp.dot(uf_r[...], wu_r[...], preferred_element_type=F32)
    ue_o[...] = jnp.maximum(ue + bu_r[...][None, :], 0.0)
    ie = jnp.dot(it_r[...], wi_r[...], preferred_element_type=F32)
    ie_o[...] = jnp.maximum(ie + bi_r[...][None, :], 0.0)

  nu, ni = uf.shape[0], itf.shape[0]
  return pl.pallas_call(
      body,
      out_shape=(_sds((nu, DIM)), _sds((ni, DIM))),
  )(uf, itf, Wu, bu, Wi, bi)


def _scale_from(parts_ref, floor=1.0):
  deg = jnp.maximum(jnp.sum(parts_ref[...], axis=0), floor)
  return lax.rsqrt(deg)


def _pad_rows(x):
  return jnp.concatenate([x, jnp.zeros((N_PAD - N_NODES, DIM), F32)], axis=0)


def _tc_prescale1(dout, ue, ie):
  def body(dout_r, ue_r, ie_r, ns_o):
    so = _scale_from(dout_r)
    nodes = jnp.concatenate([ue_r[...], ie_r[...]], axis=0)
    ns_o[...] = _pad_rows(nodes * so[:, None])

  return pl.pallas_call(body, out_shape=_sds((N_PAD, DIM)))(dout, ue, ie)


def _tc_combine1(parts, dout, din, ue, ie):
  nu = ue.shape[0]

  def body(p_r, dout_r, din_r, ue_r, ie_r, ue1_o, ie1_o, ns2_o):
    si = _scale_from(din_r)
    agg = (p_r[0] + p_r[1]) * si[:, None]
    ue1_o[...] = ue_r[...] + agg[:nu] * 0.5
    ie1_o[...] = ie_r[...] + agg[nu:] * 0.5
    so = _scale_from(dout_r)
    ns2_o[...] = _pad_rows(agg * so[:, None])

  return pl.pallas_call(
      body,
      out_shape=(_sds((nu, DIM)), _sds((N_NODES - nu, DIM)),
                 _sds((N_PAD, DIM))),
  )(parts, dout, din, ue, ie)


def _tc_combine2(parts, din, ue1, ie1, side):
  nu = ue1.shape[0]

  def body(p_r, din_r, ue_r, ie_r, side_r, ueo, ieo):
    si = _scale_from(din_r)
    agg = (p_r[0] + p_r[1]) * si[:, None]
    ueo[...] = ue_r[...] + agg[:nu] * (1.0 / 3.0)
    ieo[...] = ie_r[...] + agg[nu:] * (1.0 / 3.0) + side_r[...]

  return pl.pallas_call(
      body,
      out_shape=(_sds((nu, DIM)), _sds((N_NODES - nu, DIM))),
  )(parts, din, ue1, ie1, side)


def _tc_decoder(gathered, T_f, T_cf, W1, b1, W2):
  def body(g_r, tf_r, tcf_r, w1_r, b1_r, w2_r, lf_o, lcf_o):
    ue = g_r[0]
    z = jnp.concatenate([ue * g_r[1], ue * g_r[2]], axis=0)
    w1a = w1_r[:DIM, :]
    w1t = w1_r[DIM, :]
    base = jnp.dot(z, w1a, preferred_element_type=F32) + b1_r[...][None, :]
    for t_r, out in ((tf_r, lf_o), (tcf_r, lcf_o)):
      h = base + t_r[...][:, None] * w1t[None, :]
      h = jnp.where(h > 0, h, jnp.exp(jnp.minimum(h, 0.0)) - 1.0)
      out[...] = jnp.dot(h, w2_r[...], preferred_element_type=F32)[:, 0]

  return pl.pallas_call(
      body,
      out_shape=(_sds((2 * BDEC,)), _sds((2 * BDEC,))),
  )(gathered, T_f, T_cf, W1, b1, W2)


def kernel(user_features, item_features, item_side_feat, edge_index, userId,
           pos_itemId, neg_itemId, T_f, T_cf, Wu, bu, Wi, bi, W1, b1, W2):
  src = edge_index[0]
  dst = edge_index[1]

  def chunked(ids, pad_val):
    w = ids.reshape(NW, EPW)
    w = jnp.pad(w, ((0, 0), (0, EPP - EPW)), constant_values=pad_val)
    return w.reshape(-1)

  srcp = chunked(src, N_NODES)
  dstp = chunked(dst, 0)
  znode = jnp.zeros((N_NODES, DIM), F32)

  dout, din = _sc_degrees(src, dst)

  ue, ie = _tc_encoder(user_features, item_features, Wu, bu, Wi, bi)

  ns1 = _tc_prescale1(dout, ue, ie)
  parts1 = _sc_aggregate(ns1, srcp, dstp, znode)
  ue1, ie1, ns2 = _tc_combine1(parts1, dout, din, ue, ie)

  parts2 = _sc_aggregate(ns2, srcp, dstp, znode)
  user_embed, item_embed = _tc_combine2(parts2, din, ue1, ie1, item_side_feat)

  gathered = _sc_decoder_gather(user_embed, item_embed, userId, pos_itemId,
                                neg_itemId)
  logits_f, logits_cf = _tc_decoder(gathered, T_f, T_cf, W1, b1, W2)

  return (user_embed, item_embed, logits_f, logits_cf)


# speedup vs baseline: 1.6419x; 1.6419x over previous
import functools

import jax
import jax.numpy as jnp
from jax import lax
from jax.experimental import pallas as pl
from jax.experimental.pallas import tpu as pltpu
from jax.experimental.pallas import tpu_sc as plsc

F32 = jnp.float32

NC = 2
NS = 16
NW = NC * NS

N_NODES = 10000
DIM = 128
E = 320000
EPW = E // NW
CH = 80
NCH = EPW // CH
ROWS_PS = 640
BDEC = 8192
DEC_PW = BDEC // NW


def _sds(shape, dtype=F32):
  return jax.ShapeDtypeStruct(shape, dtype)


def _row_start(sid):
  return lax.min(sid * ROWS_PS, N_NODES - ROWS_PS)


def _sc_degrees(src, dst):
  mesh = plsc.VectorSubcoreMesh(core_axis_name="c", subcore_axis_name="s")

  @functools.partial(
      pl.kernel,
      out_type=(_sds((NW, N_NODES)), _sds((NW, N_NODES))),
      mesh=mesh,
      scratch_types=[
          pltpu.VMEM((EPW,), jnp.int32),
          pltpu.VMEM((EPW,), jnp.int32),
          pltpu.VMEM((N_NODES,), F32),
          pltpu.VMEM((N_NODES,), F32),
      ],
      compiler_params=pltpu.CompilerParams(needs_layout_passes=False),
  )
  def deg_kernel(src_hbm, dst_hbm, dout_hbm, din_hbm, sidx, didx, ho, hi):
    cid = lax.axis_index("c")
    sid = lax.axis_index("s")
    wid = sid * NC + cid
    pltpu.sync_copy(src_hbm.at[pl.ds(wid * EPW, EPW)], sidx)
    pltpu.sync_copy(dst_hbm.at[pl.ds(wid * EPW, EPW)], didx)

    zero16 = jnp.zeros((16,), F32)

    def zbody(i, _):
      ho[pl.ds(i * 16, 16)] = zero16
      hi[pl.ds(i * 16, 16)] = zero16
      return ()

    lax.fori_loop(0, N_NODES // 16, zbody, ())

    ones16 = jnp.ones((16,), F32)

    def body(i, _):
      plsc.addupdate_scatter(ho, [sidx[pl.ds(i * 16, 16)]], ones16)
      plsc.addupdate_scatter(hi, [didx[pl.ds(i * 16, 16)]], ones16)
      return ()

    lax.fori_loop(0, EPW // 16, body, ())
    pltpu.sync_copy(ho, dout_hbm.at[wid])
    pltpu.sync_copy(hi, din_hbm.at[wid])

  return deg_kernel(src, dst)


CHP = 88
EPP = 10032
NCHP = EPP // CHP
N_PAD = N_NODES + 16


def _sc_aggregate(node_pad, srcp, dstp, znode):
  mesh = plsc.VectorSubcoreMesh(core_axis_name="c", subcore_axis_name="s")

  @functools.partial(
      pl.kernel,
      out_type=_sds((NC, N_NODES, DIM)),
      mesh=mesh,
      scratch_types=[
          pltpu.VMEM((CHP,), jnp.int32),
          pltpu.VMEM((CHP,), jnp.int32),
          pltpu.VMEM((CHP,), jnp.int32),
          pltpu.VMEM((CHP,), jnp.int32),
          pltpu.VMEM((CHP, DIM), F32),
          pltpu.VMEM_SHARED((N_NODES, DIM), F32),
          pltpu.SemaphoreType.DMA,
          pltpu.SemaphoreType.DMA,
          pltpu.SemaphoreType.DMA,
      ],
  )
  def agg_kernel(nodef, srcp_h, dstp_h, znode_h, out_hbm,
                 sidx_a, didx_a, sidx_b, didx_b, rows, acc, sem, is_a, is_b):
    cid = lax.axis_index("c")
    sid = lax.axis_index("s")
    wid = sid * NC + cid
    rs = _row_start(sid)
    pltpu.sync_copy(znode_h.at[pl.ds(rs, ROWS_PS)], acc.at[pl.ds(rs, ROWS_PS)])
    plsc.subcore_barrier()

    ebase = wid * EPP
    last_off = ebase + (NCHP - 1) * CHP

    def fire_pair(off, si, di, s):
      pltpu.async_copy(srcp_h.at[pl.ds(off, CHP)], si, s)
      pltpu.async_copy(dstp_h.at[pl.ds(off, CHP)], di, s)

    def wait_pair(off, si, di, s):
      pltpu.make_async_copy(srcp_h.at[pl.ds(off, CHP)], si, s).wait()
      pltpu.make_async_copy(dstp_h.at[pl.ds(off, CHP)], di, s).wait()

    fire_pair(ebase, sidx_a, didx_a, is_a)

    def body(j, _):
      off_e = ebase + (2 * j) * CHP
      off_o = off_e + CHP
      wait_pair(off_e, sidx_a, didx_a, is_a)
      fire_pair(off_o, sidx_b, didx_b, is_b)
      pltpu.async_copy(nodef.at[sidx_a], rows, sem).wait()
      pltpu.sync_copy(rows, acc.at[didx_a], add=True)
      wait_pair(off_o, sidx_b, didx_b, is_b)
      fire_pair(lax.min(off_o + CHP, last_off), sidx_a, didx_a, is_a)
      pltpu.async_copy(nodef.at[sidx_b], rows, sem).wait()
      pltpu.sync_copy(rows, acc.at[didx_b], add=True)
      return ()

    lax.fori_loop(0, NCHP // 2, body, ())
    wait_pair(ebase, sidx_a, didx_a, is_a)
    plsc.subcore_barrier()
    pltpu.sync_copy(acc.at[pl.ds(rs, ROWS_PS)],
                    out_hbm.at[cid, pl.ds(rs, ROWS_PS)])

  return agg_kernel(node_pad, srcp, dstp, znode)


def _sc_decoder_gather(user_embed, item_embed, user_id, pos_id, neg_id):
  mesh = plsc.VectorSubcoreMesh(core_axis_name="c", subcore_axis_name="s")
  CHD = 128
  NCHD = DEC_PW // CHD

  @functools.partial(
      pl.kernel,
      out_type=_sds((3, BDEC, DIM)),
      mesh=mesh,
      scratch_types=[
          pltpu.VMEM((CHD,), jnp.int32),
          pltpu.VMEM((CHD, DIM), F32),
          pltpu.SemaphoreType.DMA,
      ],
  )
  def gather_kernel(ue_hbm, ie_hbm, uid_hbm, pid_hbm, nid_hbm, out_hbm,
                    idx, rows, sem):
    cid = lax.axis_index("c")
    sid = lax.axis_index("s")
    wid = sid * NC + cid
    base = wid * DEC_PW
    for t, (tab, ids) in enumerate(
        ((ue_hbm, uid_hbm), (ie_hbm, pid_hbm), (ie_hbm, nid_hbm))):
      for c in range(NCHD):
        off = base + c * CHD
        pltpu.sync_copy(ids.at[pl.ds(off, CHD)], idx)
        pltpu.async_copy(tab.at[idx], rows, sem).wait()
        pltpu.sync_copy(rows, out_hbm.at[t, pl.ds(off, CHD)])

  return gather_kernel(user_embed, item_embed, user_id, pos_id, neg_id)


def _tc_encoder(uf, itf, Wu, bu, Wi, bi):
  def body(uf_r, it_r, wu_r, bu_r, wi_r, bi_r, ue_o, ie_o):
    ue = jnp.dot(uf_r[...], wu_r[...], preferred_element_type=F32)
    ue_o[...] = jnp.maximum(ue + bu_r[...][None, :], 0.0)
    ie = jnp.dot(it_r[...], wi_r[...], preferred_element_type=F32)
    ie_o[...] = jnp.maximum(ie + bi_r[...][None, :], 0.0)

  nu, ni = uf.shape[0], itf.shape[0]
  return pl.pallas_call(
      body,
      out_shape=(_sds((nu, DIM)), _sds((ni, DIM))),
  )(uf, itf, Wu, bu, Wi, bi)


def _scale_from(parts_ref, floor=1.0):
  deg = jnp.maximum(jnp.sum(parts_ref[...], axis=0), floor)
  return lax.rsqrt(deg)


def _pad_rows(x):
  return jnp.concatenate([x, jnp.zeros((N_PAD - N_NODES, DIM), F32)], axis=0)


def _tc_prescale1(dout, ue, ie):
  def body(dout_r, ue_r, ie_r, ns_o):
    so = _scale_from(dout_r)
    nodes = jnp.concatenate([ue_r[...], ie_r[...]], axis=0)
    ns_o[...] = _pad_rows(nodes * so[:, None])

  return pl.pallas_call(body, out_shape=_sds((N_PAD, DIM)))(dout, ue, ie)


def _tc_combine1(parts, dout, din, ue, ie):
  nu = ue.shape[0]

  def body(p_r, dout_r, din_r, ue_r, ie_r, ue1_o, ie1_o, ns2_o):
    si = _scale_from(din_r)
    agg = (p_r[0] + p_r[1]) * si[:, None]
    ue1_o[...] = ue_r[...] + agg[:nu] * 0.5
    ie1_o[...] = ie_r[...] + agg[nu:] * 0.5
    so = _scale_from(dout_r)
    ns2_o[...] = _pad_rows(agg * so[:, None])

  return pl.pallas_call(
      body,
      out_shape=(_sds((nu, DIM)), _sds((N_NODES - nu, DIM)),
                 _sds((N_PAD, DIM))),
  )(parts, dout, din, ue, ie)


def _tc_combine2(parts, din, ue1, ie1, side):
  nu = ue1.shape[0]

  def body(p_r, din_r, ue_r, ie_r, side_r, ueo, ieo):
    si = _scale_from(din_r)
    agg = (p_r[0] + p_r[1]) * si[:, None]
    ueo[...] = ue_r[...] + agg[:nu] * (1.0 / 3.0)
    ieo[...] = ie_r[...] + agg[nu:] * (1.0 / 3.0) + side_r[...]

  return pl.pallas_call(
      body,
      out_shape=(_sds((nu, DIM)), _sds((N_NODES - nu, DIM))),
  )(parts, din, ue1, ie1, side)


def _tc_decoder(gathered, T_f, T_cf, W1, b1, W2):
  def body(g_r, tf_r, tcf_r, w1_r, b1_r, w2_r, lf_o, lcf_o):
    ue = g_r[0]
    z = jnp.concatenate([ue * g_r[1], ue * g_r[2]], axis=0)
    w1a = w1_r[:DIM, :]
    w1t = w1_r[DIM, :]
    base = jnp.dot(z, w1a, preferred_element_type=F32) + b1_r[...][None, :]
    for t_r, out in ((tf_r, lf_o), (tcf_r, lcf_o)):
      h = base + t_r[...][:, None] * w1t[None, :]
      h = jnp.where(h > 0, h, jnp.exp(jnp.minimum(h, 0.0)) - 1.0)
      out[...] = jnp.dot(h, w2_r[...], preferred_element_type=F32)[:, 0]

  return pl.pallas_call(
      body,
      out_shape=(_sds((2 * BDEC,)), _sds((2 * BDEC,))),
  )(gathered, T_f, T_cf, W1, b1, W2)


def kernel(user_features, item_features, item_side_feat, edge_index, userId,
           pos_itemId, neg_itemId, T_f, T_cf, Wu, bu, Wi, bi, W1, b1, W2):
  src = edge_index[0]
  dst = edge_index[1]

  def chunked(ids, pad_val):
    w = ids.reshape(NW, EPW)
    w = jnp.pad(w, ((0, 0), (0, EPP - EPW)), constant_values=pad_val)
    return w.reshape(-1)

  srcp = chunked(src, N_NODES)
  dstp = chunked(dst, 0)
  znode = jnp.zeros((N_NODES, DIM), F32)

  dout, din = _sc_degrees(src, dst)

  ue, ie = _tc_encoder(user_features, item_features, Wu, bu, Wi, bi)

  ns1 = _tc_prescale1(dout, ue, ie)
  parts1 = _sc_aggregate(ns1, srcp, dstp, znode)
  ue1, ie1, ns2 = _tc_combine1(parts1, dout, din, ue, ie)

  parts2 = _sc_aggregate(ns2, srcp, dstp, znode)
  user_embed, item_embed = _tc_combine2(parts2, din, ue1, ie1, item_side_feat)

  gathered = _sc_decoder_gather(user_embed, item_embed, userId, pos_itemId,
                                neg_itemId)
  logits_f, logits_cf = _tc_decoder(gathered, T_f, T_cf, W1, b1, W2)

  return (user_embed, item_embed, logits_f, logits_cf)
